# baseline (speedup 1.0000x reference)
import jax
import jax.numpy as jnp
from jax import lax
from jax.experimental import pallas as pl
from jax.experimental.pallas import tpu as pltpu
from jax.experimental.pallas import tpu_sc as plsc

_KGE_DIM = 128
_DM = 768
_B = 16384

_info = plsc.get_sparse_core_info()
_NC, _NS = _info.num_cores, _info.num_subcores
_NW = _NC * _NS
_NSLICE = 4
_BC = _B // _NSLICE
_ROWS_PER_W = _BC // _NW
_CHUNK = 128
_NCHUNK = _ROWS_PER_W // _CHUNK
_NJOBS = 3 * _NCHUNK


_NBUF = 4


def _sc_gather_body(idx_hbm, ent_hbm, rel_hbm, out_hbm,
                    idx_v, rows_v, gsems, wsems):
    wid = lax.axis_index("s") * _NC + lax.axis_index("c")
    base = wid * _ROWS_PER_W
    for f in range(3):
        pltpu.sync_copy(idx_hbm.at[f, wid], idx_v.at[f])

    tables = (ent_hbm, rel_hbm, ent_hbm)

    def gather(k):
        f, c = divmod(k, _NCHUNK)
        j = k % _NBUF
        return pltpu.async_copy(
            tables[f].at[idx_v.at[f, c]], rows_v.at[j], gsems[j])

    def writeback(k):
        f, c = divmod(k, _NCHUNK)
        j = k % _NBUF
        return pltpu.async_copy(
            rows_v.at[j], out_hbm.at[f, pl.ds(base + c * _CHUNK, _CHUNK)],
            wsems[j])

    gdescs = [None] * _NJOBS
    wdescs = [None] * _NJOBS
    for k in range(min(_NBUF, _NJOBS)):
        gdescs[k] = gather(k)
    for k in range(_NJOBS):
        gdescs[k].wait()
        wdescs[k] = writeback(k)
        nxt = k + _NBUF
        if nxt < _NJOBS:
            wdescs[k].wait()
            gdescs[nxt] = gather(nxt)
    for k in range(max(0, _NJOBS - _NBUF), _NJOBS):
        wdescs[k].wait()


def _sc_gather(idx_slice, ent_emb, rel_emb):
    mesh = plsc.VectorSubcoreMesh(core_axis_name="c", subcore_axis_name="s")
    return pl.kernel(
        _sc_gather_body,
        out_type=jax.ShapeDtypeStruct((3, _BC, _KGE_DIM), jnp.float32),
        mesh=mesh,
        scratch_types=[
            pltpu.VMEM((3, _NCHUNK, _CHUNK), jnp.int32),
            pltpu.VMEM((_NBUF, _CHUNK, _KGE_DIM), jnp.float32),
            [pltpu.SemaphoreType.DMA] * _NBUF,
            [pltpu.SemaphoreType.DMA] * _NBUF,
        ],
    )(idx_slice, ent_emb, rel_emb)


_BM = 1024


def _tc_matmul_body(emb_ref, w_ref, b_ref, out_ref):
    w = w_ref[...]
    bias = b_ref[...]
    for f in range(3):
        acc = jnp.dot(emb_ref[f], w, preferred_element_type=jnp.float32)
        out_ref[f] = acc + bias


def _tc_matmul_body_acc(prev_ref, emb_ref, w_ref, b_ref, out_ref):
    del prev_ref
    _tc_matmul_body(emb_ref, w_ref, b_ref, out_ref)


def _tc_matmul_slice(s, embs_s, W, b, prev=None):
    base_blk = s * (_BC // _BM)
    emb_spec = pl.BlockSpec((3, _BM, _KGE_DIM), lambda i: (0, i, 0))
    w_spec = pl.BlockSpec((_KGE_DIM, _DM), lambda i: (0, 0))
    b_spec = pl.BlockSpec((_DM,), lambda i: (0,))
    out_spec = pl.BlockSpec(
        (3, _BM, _DM), lambda i: (0, base_blk + i, 0))
    out_shape = jax.ShapeDtypeStruct((3, _B, _DM), jnp.float32)
    if prev is None:
        return pl.pallas_call(
            _tc_matmul_body,
            grid=(_BC // _BM,),
            in_specs=[emb_spec, w_spec, b_spec],
            out_specs=out_spec,
            out_shape=out_shape,
        )(embs_s, W, b)
    return pl.pallas_call(
        _tc_matmul_body_acc,
        grid=(_BC // _BM,),
        in_specs=[
            pl.BlockSpec(memory_space=pl.ANY),
            emb_spec, w_spec, b_spec,
        ],
        out_specs=out_spec,
        out_shape=out_shape,
        input_output_aliases={0: 0},
    )(prev, embs_s, W, b)


def kernel(triples, ent_emb, rel_emb, W, b):
    idx_all = jnp.transpose(triples, (1, 0)).reshape(
        3, _NSLICE, _NW, _NCHUNK, _CHUNK)
    embs = [_sc_gather(idx_all[:, s], ent_emb, rel_emb)
            for s in range(_NSLICE)]
    out = None
    for s in range(_NSLICE):
        out = _tc_matmul_slice(s, embs[s], W, b, prev=out)
    return jnp.transpose(out, (1, 0, 2))


# speedup vs baseline: 1.0126x; 1.0126x over previous
import jax
import jax.numpy as jnp
from jax import lax
from jax.experimental import pallas as pl
from jax.experimental.pallas import tpu as pltpu
from jax.experimental.pallas import tpu_sc as plsc

_KGE_DIM = 128
_DM = 768
_B = 16384

_info = plsc.get_sparse_core_info()
_NC, _NS = _info.num_cores, _info.num_subcores
_NW = _NC * _NS
_NSLICE = 2
_BC = _B // _NSLICE
_ROWS_PER_W = _BC // _NW
_CHUNK = 128
_NCHUNK = _ROWS_PER_W // _CHUNK
_NJOBS = 3 * _NCHUNK


_NBUF = 4


def _sc_gather_body(idx_hbm, ent_hbm, rel_hbm, out_hbm,
                    idx_v, rows_v, gsems, wsems):
    wid = lax.axis_index("s") * _NC + lax.axis_index("c")
    base = wid * _ROWS_PER_W
    for f in range(3):
        pltpu.sync_copy(idx_hbm.at[f, wid], idx_v.at[f])

    tables = (ent_hbm, rel_hbm, ent_hbm)

    def gather(k):
        f, c = divmod(k, _NCHUNK)
        j = k % _NBUF
        return pltpu.async_copy(
            tables[f].at[idx_v.at[f, c]], rows_v.at[j], gsems[j])

    def writeback(k):
        f, c = divmod(k, _NCHUNK)
        j = k % _NBUF
        return pltpu.async_copy(
            rows_v.at[j], out_hbm.at[f, pl.ds(base + c * _CHUNK, _CHUNK)],
            wsems[j])

    gdescs = [None] * _NJOBS
    wdescs = [None] * _NJOBS
    for k in range(min(_NBUF, _NJOBS)):
        gdescs[k] = gather(k)
    for k in range(_NJOBS):
        gdescs[k].wait()
        wdescs[k] = writeback(k)
        nxt = k + _NBUF
        if nxt < _NJOBS:
            wdescs[k].wait()
            gdescs[nxt] = gather(nxt)
    for k in range(max(0, _NJOBS - _NBUF), _NJOBS):
        wdescs[k].wait()


def _sc_gather(idx_slice, ent_emb, rel_emb):
    mesh = plsc.VectorSubcoreMesh(core_axis_name="c", subcore_axis_name="s")
    return pl.kernel(
        _sc_gather_body,
        out_type=jax.ShapeDtypeStruct((3, _BC, _KGE_DIM), jnp.float32),
        mesh=mesh,
        scratch_types=[
            pltpu.VMEM((3, _NCHUNK, _CHUNK), jnp.int32),
            pltpu.VMEM((_NBUF, _CHUNK, _KGE_DIM), jnp.float32),
            [pltpu.SemaphoreType.DMA] * _NBUF,
            [pltpu.SemaphoreType.DMA] * _NBUF,
        ],
    )(idx_slice, ent_emb, rel_emb)


_BM = 1024


def _tc_matmul_body(emb_ref, w_ref, b_ref, out_ref):
    w = w_ref[...]
    bias = b_ref[...]
    for f in range(3):
        acc = jnp.dot(emb_ref[f], w, preferred_element_type=jnp.float32)
        out_ref[f] = acc + bias


def _tc_matmul_body_acc(prev_ref, emb_ref, w_ref, b_ref, out_ref):
    del prev_ref
    _tc_matmul_body(emb_ref, w_ref, b_ref, out_ref)


def _tc_matmul_slice(s, embs_s, W, b, prev=None):
    base_blk = s * (_BC // _BM)
    emb_spec = pl.BlockSpec((3, _BM, _KGE_DIM), lambda i: (0, i, 0))
    w_spec = pl.BlockSpec((_KGE_DIM, _DM), lambda i: (0, 0))
    b_spec = pl.BlockSpec((_DM,), lambda i: (0,))
    out_spec = pl.BlockSpec(
        (3, _BM, _DM), lambda i: (0, base_blk + i, 0))
    out_shape = jax.ShapeDtypeStruct((3, _B, _DM), jnp.float32)
    if prev is None:
        return pl.pallas_call(
            _tc_matmul_body,
            grid=(_BC // _BM,),
            in_specs=[emb_spec, w_spec, b_spec],
            out_specs=out_spec,
            out_shape=out_shape,
        )(embs_s, W, b)
    return pl.pallas_call(
        _tc_matmul_body_acc,
        grid=(_BC // _BM,),
        in_specs=[
            pl.BlockSpec(memory_space=pl.ANY),
            emb_spec, w_spec, b_spec,
        ],
        out_specs=out_spec,
        out_shape=out_shape,
        input_output_aliases={0: 0},
    )(prev, embs_s, W, b)


def kernel(triples, ent_emb, rel_emb, W, b):
    idx_all = jnp.transpose(triples, (1, 0)).reshape(
        3, _NSLICE, _NW, _NCHUNK, _CHUNK)
    embs = [_sc_gather(idx_all[:, s], ent_emb, rel_emb)
            for s in range(_NSLICE)]
    out = None
    for s in range(_NSLICE):
        out = _tc_matmul_slice(s, embs[s], W, b, prev=out)
    return jnp.transpose(out, (1, 0, 2))


# speedup vs baseline: 1.0362x; 1.0233x over previous
import jax
import jax.numpy as jnp
from jax import lax
from jax.experimental import pallas as pl
from jax.experimental.pallas import tpu as pltpu
from jax.experimental.pallas import tpu_sc as plsc

_KGE_DIM = 128
_DM = 768
_B = 16384

_info = plsc.get_sparse_core_info()
_NC, _NS = _info.num_cores, _info.num_subcores
_NW = _NC * _NS
_NSLICE = 1
_BC = _B // _NSLICE
_ROWS_PER_W = _BC // _NW
_CHUNK = 128
_NCHUNK = _ROWS_PER_W // _CHUNK
_NJOBS = 3 * _NCHUNK


_NBUF = 4


def _sc_gather_body(idx_hbm, ent_hbm, rel_hbm, out_hbm,
                    idx_v, rows_v, gsems, wsems):
    wid = lax.axis_index("s") * _NC + lax.axis_index("c")
    base = wid * _ROWS_PER_W
    for f in range(3):
        pltpu.sync_copy(idx_hbm.at[f, wid], idx_v.at[f])

    tables = (ent_hbm, rel_hbm, ent_hbm)

    def gather(k):
        f, c = divmod(k, _NCHUNK)
        j = k % _NBUF
        return pltpu.async_copy(
            tables[f].at[idx_v.at[f, c]], rows_v.at[j], gsems[j])

    def writeback(k):
        f, c = divmod(k, _NCHUNK)
        j = k % _NBUF
        return pltpu.async_copy(
            rows_v.at[j], out_hbm.at[f, pl.ds(base + c * _CHUNK, _CHUNK)],
            wsems[j])

    gdescs = [None] * _NJOBS
    wdescs = [None] * _NJOBS
    for k in range(min(_NBUF, _NJOBS)):
        gdescs[k] = gather(k)
    for k in range(_NJOBS):
        gdescs[k].wait()
        wdescs[k] = writeback(k)
        nxt = k + _NBUF
        if nxt < _NJOBS:
            wdescs[k].wait()
            gdescs[nxt] = gather(nxt)
    for k in range(max(0, _NJOBS - _NBUF), _NJOBS):
        wdescs[k].wait()


def _sc_gather(idx_slice, ent_emb, rel_emb):
    mesh = plsc.VectorSubcoreMesh(core_axis_name="c", subcore_axis_name="s")
    return pl.kernel(
        _sc_gather_body,
        out_type=jax.ShapeDtypeStruct((3, _BC, _KGE_DIM), jnp.float32),
        mesh=mesh,
        scratch_types=[
            pltpu.VMEM((3, _NCHUNK, _CHUNK), jnp.int32),
            pltpu.VMEM((_NBUF, _CHUNK, _KGE_DIM), jnp.float32),
            [pltpu.SemaphoreType.DMA] * _NBUF,
            [pltpu.SemaphoreType.DMA] * _NBUF,
        ],
    )(idx_slice, ent_emb, rel_emb)


_BM = 2048


def _tc_matmul_body(emb_ref, w_ref, b_ref, out_ref):
    w = w_ref[...]
    bias = b_ref[...]
    for f in range(3):
        acc = jnp.dot(emb_ref[f], w, preferred_element_type=jnp.float32)
        out_ref[f] = acc + bias


def _tc_matmul_body_acc(prev_ref, emb_ref, w_ref, b_ref, out_ref):
    del prev_ref
    _tc_matmul_body(emb_ref, w_ref, b_ref, out_ref)


def _tc_matmul_slice(s, embs_s, W, b, prev=None):
    base_blk = s * (_BC // _BM)
    emb_spec = pl.BlockSpec((3, _BM, _KGE_DIM), lambda i: (0, i, 0))
    w_spec = pl.BlockSpec((_KGE_DIM, _DM), lambda i: (0, 0))
    b_spec = pl.BlockSpec((_DM,), lambda i: (0,))
    out_spec = pl.BlockSpec(
        (3, _BM, _DM), lambda i: (0, base_blk + i, 0))
    out_shape = jax.ShapeDtypeStruct((3, _B, _DM), jnp.float32)
    if prev is None:
        return pl.pallas_call(
            _tc_matmul_body,
            grid=(_BC // _BM,),
            in_specs=[emb_spec, w_spec, b_spec],
            out_specs=out_spec,
            out_shape=out_shape,
        )(embs_s, W, b)
    return pl.pallas_call(
        _tc_matmul_body_acc,
        grid=(_BC // _BM,),
        in_specs=[
            pl.BlockSpec(memory_space=pl.ANY),
            emb_spec, w_spec, b_spec,
        ],
        out_specs=out_spec,
        out_shape=out_shape,
        input_output_aliases={0: 0},
    )(prev, embs_s, W, b)


def kernel(triples, ent_emb, rel_emb, W, b):
    idx_all = jnp.transpose(triples, (1, 0)).reshape(
        3, _NSLICE, _NW, _NCHUNK, _CHUNK)
    embs = [_sc_gather(idx_all[:, s], ent_emb, rel_emb)
            for s in range(_NSLICE)]
    out = None
    for s in range(_NSLICE):
        out = _tc_matmul_slice(s, embs[s], W, b, prev=out)
    return jnp.transpose(out, (1, 0, 2))


# speedup vs baseline: 1.0553x; 1.0184x over previous
import jax
import jax.numpy as jnp
from jax import lax
from jax.experimental import pallas as pl
from jax.experimental.pallas import tpu as pltpu
from jax.experimental.pallas import tpu_sc as plsc

_KGE_DIM = 128
_DM = 768
_B = 16384

_info = plsc.get_sparse_core_info()
_NC, _NS = _info.num_cores, _info.num_subcores
_NW = _NC * _NS
_NSLICE = 1
_BC = _B // _NSLICE
_ROWS_PER_W = _BC // _NW
_CHUNK = 128
_NCHUNK = _ROWS_PER_W // _CHUNK
_NJOBS = 3 * _NCHUNK


_NBUF = 6


def _sc_gather_body(idx_hbm, ent_hbm, rel_hbm, out_hbm,
                    idx_v, rows_v, gsems, wsems):
    wid = lax.axis_index("s") * _NC + lax.axis_index("c")
    base = wid * _ROWS_PER_W
    idescs = [
        pltpu.async_copy(idx_hbm.at[f, wid], idx_v.at[f], gsems[f])
        for f in range(3)
    ]
    for d in idescs:
        d.wait()

    tables = (ent_hbm, rel_hbm, ent_hbm)

    def gather(k):
        f, c = divmod(k, _NCHUNK)
        j = k % _NBUF
        return pltpu.async_copy(
            tables[f].at[idx_v.at[f, c]], rows_v.at[j], gsems[j])

    def writeback(k):
        f, c = divmod(k, _NCHUNK)
        j = k % _NBUF
        return pltpu.async_copy(
            rows_v.at[j], out_hbm.at[f, pl.ds(base + c * _CHUNK, _CHUNK)],
            wsems[j])

    gdescs = [None] * _NJOBS
    wdescs = [None] * _NJOBS
    for k in range(min(_NBUF, _NJOBS)):
        gdescs[k] = gather(k)
    for k in range(_NJOBS):
        gdescs[k].wait()
        wdescs[k] = writeback(k)
        nxt = k + _NBUF
        if nxt < _NJOBS:
            wdescs[k].wait()
            gdescs[nxt] = gather(nxt)
    for k in range(max(0, _NJOBS - _NBUF), _NJOBS):
        wdescs[k].wait()


def _sc_gather(idx_slice, ent_emb, rel_emb):
    mesh = plsc.VectorSubcoreMesh(core_axis_name="c", subcore_axis_name="s")
    return pl.kernel(
        _sc_gather_body,
        out_type=jax.ShapeDtypeStruct((3, _BC, _KGE_DIM), jnp.float32),
        mesh=mesh,
        scratch_types=[
            pltpu.VMEM((3, _NCHUNK, _CHUNK), jnp.int32),
            pltpu.VMEM((_NBUF, _CHUNK, _KGE_DIM), jnp.float32),
            [pltpu.SemaphoreType.DMA] * _NBUF,
            [pltpu.SemaphoreType.DMA] * _NBUF,
        ],
    )(idx_slice, ent_emb, rel_emb)


_BM = 2048


def _tc_matmul_body(emb_ref, w_ref, b_ref, out_ref):
    w = w_ref[...]
    bias = b_ref[...]
    for f in range(3):
        acc = jnp.dot(emb_ref[f], w, preferred_element_type=jnp.float32)
        out_ref[f] = acc + bias


def _tc_matmul_body_acc(prev_ref, emb_ref, w_ref, b_ref, out_ref):
    del prev_ref
    _tc_matmul_body(emb_ref, w_ref, b_ref, out_ref)


def _tc_matmul_slice(s, embs_s, W, b, prev=None):
    base_blk = s * (_BC // _BM)
    emb_spec = pl.BlockSpec((3, _BM, _KGE_DIM), lambda i: (0, i, 0))
    w_spec = pl.BlockSpec((_KGE_DIM, _DM), lambda i: (0, 0))
    b_spec = pl.BlockSpec((_DM,), lambda i: (0,))
    out_spec = pl.BlockSpec(
        (3, _BM, _DM), lambda i: (0, base_blk + i, 0))
    out_shape = jax.ShapeDtypeStruct((3, _B, _DM), jnp.float32)
    if prev is None:
        return pl.pallas_call(
            _tc_matmul_body,
            grid=(_BC // _BM,),
            in_specs=[emb_spec, w_spec, b_spec],
            out_specs=out_spec,
            out_shape=out_shape,
        )(embs_s, W, b)
    return pl.pallas_call(
        _tc_matmul_body_acc,
        grid=(_BC // _BM,),
        in_specs=[
            pl.BlockSpec(memory_space=pl.ANY),
            emb_spec, w_spec, b_spec,
        ],
        out_specs=out_spec,
        out_shape=out_shape,
        input_output_aliases={0: 0},
    )(prev, embs_s, W, b)


def kernel(triples, ent_emb, rel_emb, W, b):
    idx_all = jnp.transpose(triples, (1, 0)).reshape(
        3, _NSLICE, _NW, _NCHUNK, _CHUNK)
    embs = [_sc_gather(idx_all[:, s], ent_emb, rel_emb)
            for s in range(_NSLICE)]
    out = None
    for s in range(_NSLICE):
        out = _tc_matmul_slice(s, embs[s], W, b, prev=out)
    return jnp.transpose(out, (1, 0, 2))
